# baseline (speedup 1.0000x reference)
import numpy as np
import jax
import jax.numpy as jnp
from jax.experimental import pallas as pl
from jax.experimental.pallas import tpu as pltpu

_N = 32
_TH = 1.1
_PAD_BEGIN = (_N - 1) // 2
_PAD_END = _N - 1 - _PAD_BEGIN
_C = 256
_BR = 16


def _band_matrices():
    mm = np.zeros((_C, _C), np.float32)
    mv = np.zeros((_C, _C), np.float32)
    mp = np.zeros((_C, _C), np.float32)
    for l in range(_C):
        lo = l - _PAD_BEGIN
        hi = l + _PAD_END
        for c in range(_C):
            if lo <= c - _C <= hi:
                mm[c, l] = 1.0
            if lo <= c <= hi:
                mv[c, l] = 1.0
            if lo <= c + _C <= hi:
                mp[c, l] = 1.0
    return np.concatenate([mm, mv, mp], axis=0)


_BANDS = _band_matrices()


def _hi16(a):
    u = pltpu.bitcast(a, jnp.uint32) & jnp.uint32(0xFFFF0000)
    return pltpu.bitcast(u, jnp.float32)


def _body(x_ref, m_ref, out_ref, sig_ref):
    rows = x_ref.shape[0]
    rows_per_batch = 16384 // _C

    xv = x_ref[...]
    xm = pltpu.roll(xv, 1, axis=0)
    xp = pltpu.roll(xv, rows - 1, axis=0)

    mm = m_ref[0:_C, :]
    mv = m_ref[_C:2 * _C, :]
    mp = m_ref[2 * _C:3 * _C, :]

    def banded(vm, vv, vp):
        acc = None
        for v, m in ((vv, mv), (vm, mm), (vp, mp)):
            h = _hi16(v)
            l = v - h
            d = jnp.dot(h, m, preferred_element_type=jnp.float32)
            d = d + jnp.dot(l, m, preferred_element_type=jnp.float32)
            acc = d if acc is None else acc + d
        return acc

    s1 = banded(xm, xv, xp)
    s2 = banded(xm * xm, xv * xv, xp * xp)

    inv_n = 1.0 / _N
    mean = s1 * inv_n
    var = jnp.maximum(s2 * inv_n - mean * mean, 0.0)
    sigma = jnp.sqrt(var)

    rmod = jax.lax.broadcasted_iota(jnp.int32, xv.shape, 0) % rows_per_batch
    lane = jax.lax.broadcasted_iota(jnp.int32, xv.shape, 1)
    bad = ((rmod == 0) & (lane < _PAD_BEGIN)) | (
        (rmod == rows_per_batch - 1) & (lane >= _C - _PAD_END))
    sigma = jnp.where(bad, 0.0, sigma)

    sigma_n = sigma * (1.0 / (2.0 * _TH))
    hard = jnp.clip(jnp.round(sigma_n), 0.0, 1.0)

    sig_ref[...] = sigma
    out_ref[...] = sigma_n + (hard - sigma_n)


@jax.jit
def kernel(input_attenuation):
    x = input_attenuation
    B, T = x.shape
    rows_per_batch = T // _C
    xr = x.reshape(B * rows_per_batch, _C)
    R = _BR * rows_per_batch
    grid = (xr.shape[0] // R,)
    spec = pl.BlockSpec((R, _C), lambda i: (i, 0))
    bands = jnp.asarray(_BANDS)
    out, sig = pl.pallas_call(
        _body,
        grid=grid,
        in_specs=[spec, pl.BlockSpec((3 * _C, _C), lambda i: (0, 0))],
        out_specs=[spec, spec],
        out_shape=[jax.ShapeDtypeStruct(xr.shape, x.dtype)] * 2,
        compiler_params=pltpu.CompilerParams(
            dimension_semantics=("parallel",),
            vmem_limit_bytes=100 * 1024 * 1024,
        ),
        name="stdwet_dry_mxu",
    )(xr, bands)
    return (out.reshape(B, T), sig.reshape(B, T))


# speedup vs baseline: 2.5919x; 2.5919x over previous
import numpy as np
import jax
import jax.numpy as jnp
from jax.experimental import pallas as pl
from jax.experimental.pallas import tpu as pltpu

_N = 32
_TH = 1.1
_PAD_BEGIN = (_N - 1) // 2
_PAD_END = _N - 1 - _PAD_BEGIN
_C = 256
_BR = 16


def _band_matrices():
    mm = np.zeros((_C, _C), np.float32)
    mv = np.zeros((_C, _C), np.float32)
    mp = np.zeros((_C, _C), np.float32)
    for l in range(_C):
        lo = l - _PAD_BEGIN
        hi = l + _PAD_END
        for c in range(_C):
            if lo <= c - _C <= hi:
                mm[c, l] = 1.0
            if lo <= c <= hi:
                mv[c, l] = 1.0
            if lo <= c + _C <= hi:
                mp[c, l] = 1.0
    return np.concatenate([mm, mv, mp], axis=0)


_BANDS = _band_matrices()


def _hi16(a):
    u = pltpu.bitcast(a, jnp.uint32) & jnp.uint32(0xFFFF0000)
    return pltpu.bitcast(u, jnp.float32)


def _body(x_ref, m_ref, out_ref, sig_ref):
    T = x_ref.shape[-1]
    n_chunks = T // _C
    rows = _BR * n_chunks

    xv = jnp.concatenate(
        [x_ref[:, i * _C:(i + 1) * _C] for i in range(n_chunks)], axis=0)
    xm = pltpu.roll(xv, _BR, axis=0)
    xp = pltpu.roll(xv, rows - _BR, axis=0)

    mm = m_ref[0:_C, :]
    mv = m_ref[_C:2 * _C, :]
    mp = m_ref[2 * _C:3 * _C, :]

    def banded(vm, vv, vp):
        acc = None
        for v, m in ((vv, mv), (vm, mm), (vp, mp)):
            h = _hi16(v)
            l = v - h
            d = jnp.dot(h, m, preferred_element_type=jnp.float32)
            d = d + jnp.dot(l, m, preferred_element_type=jnp.float32)
            acc = d if acc is None else acc + d
        return acc

    s1 = banded(xm, xv, xp)
    s2 = banded(xm * xm, xv * xv, xp * xp)

    inv_n = 1.0 / _N
    mean = s1 * inv_n
    var = jnp.maximum(s2 * inv_n - mean * mean, 0.0)
    sigma = jnp.sqrt(var)

    ridx = jax.lax.broadcasted_iota(jnp.int32, sigma.shape, 0)
    lane = jax.lax.broadcasted_iota(jnp.int32, sigma.shape, 1)
    bad = ((ridx < _BR) & (lane < _PAD_BEGIN)) | (
        (ridx >= rows - _BR) & (lane >= _C - _PAD_END))
    sigma = jnp.where(bad, 0.0, sigma)

    sigma_n = sigma * (1.0 / (2.0 * _TH))
    hard = jnp.clip(jnp.round(sigma_n), 0.0, 1.0)
    out = sigma_n + (hard - sigma_n)

    for i in range(n_chunks):
        sig_ref[:, i * _C:(i + 1) * _C] = sigma[i * _BR:(i + 1) * _BR, :]
        out_ref[:, i * _C:(i + 1) * _C] = out[i * _BR:(i + 1) * _BR, :]


@jax.jit
def kernel(input_attenuation):
    x = input_attenuation
    B, T = x.shape
    grid = (B // _BR,)
    spec = pl.BlockSpec((_BR, T), lambda i: (i, 0))
    bands = jnp.asarray(_BANDS)
    out, sig = pl.pallas_call(
        _body,
        grid=grid,
        in_specs=[spec, pl.BlockSpec((3 * _C, _C), lambda i: (0, 0))],
        out_specs=[spec, spec],
        out_shape=[jax.ShapeDtypeStruct((B, T), x.dtype)] * 2,
        compiler_params=pltpu.CompilerParams(
            dimension_semantics=("parallel",),
            vmem_limit_bytes=100 * 1024 * 1024,
        ),
        name="stdwet_dry_mxu",
    )(x, bands)
    return (out, sig)
